# baseline (speedup 1.0000x reference)
import functools

import jax
import jax.numpy as jnp
from jax import lax
from jax.experimental import pallas as pl
from jax.experimental.pallas import tpu as pltpu
from jax.experimental.pallas import tpu_sc as plsc


def kernel(x, weight):
    seq_len = x.shape[-2]
    dim = weight.shape[1]
    info = plsc.get_sparse_core_info()
    num_tiles = info.num_cores * info.num_subcores
    rows_per_tile = seq_len // num_tiles
    mesh = plsc.VectorSubcoreMesh(core_axis_name="c", subcore_axis_name="s")

    @functools.partial(
        pl.kernel,
        out_type=jax.ShapeDtypeStruct((seq_len, dim), weight.dtype),
        mesh=mesh,
        scratch_types=[pltpu.SemaphoreType.DMA],
    )
    def _stripe_copy(w_hbm, o_hbm, sem):
        tile = lax.axis_index("s") * info.num_cores + lax.axis_index("c")
        base = tile * rows_per_tile
        pltpu.async_copy(
            w_hbm.at[pl.ds(base, rows_per_tile)],
            o_hbm.at[pl.ds(base, rows_per_tile)],
            sem,
        ).wait()

    return _stripe_copy(weight)


# speedup vs baseline: 48.7415x; 48.7415x over previous
import jax
import jax.numpy as jnp
from jax.experimental import pallas as pl
from jax.experimental.pallas import tpu as pltpu


_BLOCK_ROWS = 1024


def _slice_copy_kernel(w_ref, o_ref):
    o_ref[...] = w_ref[...]


def kernel(x, weight):
    seq_len = x.shape[-2]
    dim = weight.shape[1]
    grid = (seq_len // _BLOCK_ROWS,)
    return pl.pallas_call(
        _slice_copy_kernel,
        out_shape=jax.ShapeDtypeStruct((seq_len, dim), weight.dtype),
        grid=grid,
        in_specs=[pl.BlockSpec((_BLOCK_ROWS, dim), lambda i: (i, 0))],
        out_specs=pl.BlockSpec((_BLOCK_ROWS, dim), lambda i: (i, 0)),
    )(weight)
